# baseline (speedup 1.0000x reference)
import functools
import math

import jax
import jax.numpy as jnp
from jax import lax
from jax.experimental import pallas as pl
from jax.experimental.pallas import tpu as pltpu
from jax.experimental.pallas import tpu_sc as plsc

N_ROWS = 1000000
C = 32
B = 4096
L = 200
LP = L + 1

_info = plsc.get_sparse_core_info()
NC = _info.num_cores
NS = _info.num_subcores
LANES = _info.num_lanes
NW = NC * NS
B_PER_W = B // NW

SCALE = math.sqrt(C)

_mesh = plsc.VectorSubcoreMesh(core_axis_name="c", subcore_axis_name="s")


@functools.partial(
    pl.kernel,
    mesh=_mesh,
    out_type=jax.ShapeDtypeStruct((B, C, L), jnp.float32),
    compiler_params=pltpu.CompilerParams(
        needs_layout_passes=False, use_tc_tiling_on_sc=False
    ),
    scratch_types=[
        pltpu.VMEM((B_PER_W, L), jnp.int32),
        pltpu.VMEM((L, C), jnp.float32),
        pltpu.VMEM((L, C), jnp.float32),
        pltpu.VMEM((C, LP), jnp.float32),
        pltpu.VMEM((C, LP), jnp.float32),
        pltpu.SemaphoreType.DMA,
        pltpu.SemaphoreType.DMA,
        pltpu.SemaphoreType.DMA,
        pltpu.SemaphoreType.DMA,
    ],
)
def _emb_kernel(
    x_hbm, w_hbm, out_hbm,
    idx_v, rows0, rows1, out0, out1,
    sem_g0, sem_g1, sem_o0, sem_o1,
):
    wid = lax.axis_index("s") * NC + lax.axis_index("c")
    base = wid * B_PER_W

    pltpu.sync_copy(x_hbm.at[pl.ds(base, B_PER_W)], idx_v)

    lanes_iota = lax.iota(jnp.int32, LANES)

    def fire_gather(b, rows, sem):
        pltpu.async_copy(
            w_hbm.at[idx_v.at[b, pl.ds(0, 128)]], rows.at[pl.ds(0, 128)], sem
        )
        pltpu.async_copy(
            w_hbm.at[idx_v.at[b, pl.ds(128, 72)]], rows.at[pl.ds(128, 72)], sem
        )

    def wait_gather(rows, sem):
        pltpu.make_async_copy(w_hbm.at[pl.ds(0, L)], rows, sem).wait()

    def transpose(rows, out):
        def tbody(l, carry):
            l_splat = jnp.full((LANES,), l, jnp.int32)
            for c0 in range(0, C, LANES):
                g = rows[l, pl.ds(c0, LANES)]
                plsc.store_scatter(out, [lanes_iota + c0, l_splat], g * SCALE)
            return carry

        lax.fori_loop(0, L, tbody, 0, unroll=20)

    def fire_out(out, b, sem):
        pltpu.async_copy(out.at[:, pl.ds(0, L)], out_hbm.at[base + b], sem)

    def wait_out(out, sem):
        pltpu.make_async_copy(
            out.at[:, pl.ds(0, L)], out_hbm.at[base], sem
        ).wait()

    fire_gather(0, rows0, sem_g0)
    fire_gather(1, rows1, sem_g1)

    def body(i, carry):
        b0 = 2 * i
        b1 = 2 * i + 1
        wait_gather(rows0, sem_g0)
        transpose(rows0, out0)
        fire_gather(jnp.minimum(b0 + 2, B_PER_W - 1), rows0, sem_g0)
        fire_out(out0, b0, sem_o0)
        wait_gather(rows1, sem_g1)
        transpose(rows1, out1)
        fire_gather(jnp.minimum(b1 + 2, B_PER_W - 1), rows1, sem_g1)
        fire_out(out1, b1, sem_o1)
        wait_out(out0, sem_o0)
        wait_out(out1, sem_o1)
        return carry

    lax.fori_loop(0, B_PER_W // 2, body, 0)

    wait_gather(rows0, sem_g0)
    wait_gather(rows1, sem_g1)


def kernel(x, emb_weight):
    return _emb_kernel(x.astype(jnp.int32), emb_weight)
